# baseline (speedup 1.0000x reference)
import functools

import jax
import jax.numpy as jnp
from jax import lax
from jax.experimental import pallas as pl
from jax.experimental.pallas import tpu as pltpu
from jax.experimental.pallas import tpu_sc as plsc

VOCAB = 1000000
EMB_DIM = 64
BATCH = 16384

_NUM_CORES = 2
_NUM_SUBCORES = 16
_NUM_WORKERS = _NUM_CORES * _NUM_SUBCORES
_B_PER_W = BATCH // _NUM_WORKERS
_CHUNK = 128
_NCHUNK = _B_PER_W // _CHUNK


def _make_gather_kernel():
    mesh = plsc.VectorSubcoreMesh(
        core_axis_name="c", subcore_axis_name="s", num_cores=_NUM_CORES
    )

    @functools.partial(
        pl.kernel,
        mesh=mesh,
        out_type=jax.ShapeDtypeStruct((BATCH, EMB_DIM), jnp.float32),
        scratch_types=[
            pltpu.VMEM((_NCHUNK, _CHUNK), jnp.int32),
            pltpu.VMEM((_B_PER_W, EMB_DIM), jnp.float32),
            pltpu.SemaphoreType.DMA,
        ],
        compiler_params=pltpu.CompilerParams(use_tc_tiling_on_sc=False),
    )
    def gather_kernel(table_hbm, idx_hbm, out_hbm, idx_v, rows_v, sem):
        wid = lax.axis_index("s") * _NUM_CORES + lax.axis_index("c")
        base = wid * _B_PER_W
        pltpu.sync_copy(idx_hbm.at[wid], idx_v)
        copies = []
        for j in range(_NCHUNK):
            copies.append(
                pltpu.async_copy(
                    table_hbm.at[idx_v.at[j]],
                    rows_v.at[pl.ds(j * _CHUNK, _CHUNK)],
                    sem,
                )
            )
        for c in copies:
            c.wait()
        pltpu.sync_copy(rows_v, out_hbm.at[pl.ds(base, _B_PER_W)])

    return gather_kernel


_GATHER = _make_gather_kernel()


@jax.jit
def kernel(indices, table):
    idx = indices.astype(jnp.int32).reshape(_NUM_WORKERS, _NCHUNK, _CHUNK)
    return _GATHER(table, idx)


# speedup vs baseline: 1.7306x; 1.7306x over previous
import functools

import jax
import jax.numpy as jnp
from jax import lax
from jax.experimental import pallas as pl
from jax.experimental.pallas import tpu as pltpu
from jax.experimental.pallas import tpu_sc as plsc

VOCAB = 1000000
EMB_DIM = 64
BATCH = 16384

_NC = 2
_NW = 32
_L = 16

_STRIDE_W = 31232
_COLS = 248
_CHUNK_W = 512
_NCHUNK = _COLS * 128 // _CHUNK_W
_TAIL_LO = 999936
_TAIL_W = VOCAB - _TAIL_LO
_DUMP_ROW = BATCH
_OUT_ROWS = BATCH + 128
_NIDX_G = BATCH // _L
_STAGE = 128


def _make_kernel():
    mesh = plsc.VectorSubcoreMesh(
        core_axis_name="c", subcore_axis_name="s", num_cores=_NC
    )

    @functools.partial(
        pl.kernel,
        mesh=mesh,
        out_type=jax.ShapeDtypeStruct((_OUT_ROWS, 128), jnp.float32),
        scratch_types=[
            pltpu.VMEM((BATCH + _L,), jnp.int32),
            pltpu.VMEM((BATCH + _L,), jnp.int32),
            pltpu.VMEM((EMB_DIM, _CHUNK_W), jnp.float32),
            pltpu.VMEM((EMB_DIM, _CHUNK_W), jnp.float32),
            pltpu.VMEM((EMB_DIM, _TAIL_W), jnp.float32),
            pltpu.VMEM((_STAGE, 128), jnp.float32),
            pltpu.VMEM((_STAGE,), jnp.int32),
            pltpu.VMEM((4 * _L,), jnp.int32),
            pltpu.VMEM((4 * _L,), jnp.int32),
            pltpu.SemaphoreType.DMA,
            pltpu.SemaphoreType.DMA,
            pltpu.SemaphoreType.DMA,
        ],
        compiler_params=pltpu.CompilerParams(
            use_tc_tiling_on_sc=True, needs_layout_passes=False
        ),
    )
    def fastgather(
        tableT_hbm,
        idx_hbm,
        tail_hbm,
        out_hbm,
        idx_v,
        hits_b,
        slab_a,
        slab_b,
        tail_v,
        stage,
        sidx,
        qw,
        qb,
        sem_a,
        sem_b,
        sem_s,
    ):
        wid = lax.axis_index("s") * _NC + lax.axis_index("c")
        lo = wid * _STRIDE_W
        iota = lax.iota(jnp.int32, _L)

        pltpu.sync_copy(idx_hbm, idx_v.at[pl.ds(0, BATCH)])
        hits_w = idx_v
        hi_w = jnp.where(
            wid == _NW - 1, _COLS * 128 + _TAIL_W, _COLS * 128
        ).astype(jnp.int32)

        dump = jnp.full((_L,), _DUMP_ROW, jnp.int32)
        for j in range(_STAGE // _L):
            sidx[pl.ds(j * _L, _L)] = dump

        def scan_body(g, nh):
            v = idx_v[pl.ds(g * _L, _L)]
            wl = v - lo
            m = (wl >= 0) & (wl < hi_w)

            def append(n):
                mi = m.astype(jnp.int32)
                cs = plsc.cumsum(mi)
                dest = n + cs - mi
                plsc.store_scatter(hits_w, [dest], wl, mask=m)
                plsc.store_scatter(hits_b, [dest], iota + g * _L, mask=m)
                return n + jnp.max(cs)

            return lax.cond(jnp.any(m), append, lambda n: n, nh)

        nh = lax.fori_loop(0, _NIDX_G, scan_body, jnp.int32(0))
        nsearch = lax.div(nh + (_L - 1), jnp.int32(_L))

        def extract_group(slab, qpos, sbase, mask):
            wv = qw[pl.ds(qpos, _L)]
            bv = qb[pl.ds(qpos, _L)]
            svec = sbase + iota
            plsc.store_scatter(sidx, [svec], bv, mask=mask)

            def e_body(eo, _):
                for es in range(8):
                    e = eo * 8 + es
                    ev = jnp.full((_L,), 0, jnp.int32) + e
                    vals = plsc.load_gather(slab, [ev, wv], mask=mask)
                    plsc.store_scatter(stage, [svec, ev], vals, mask=mask)
                return 0

            lax.fori_loop(0, EMB_DIM // 8, e_body, 0)

        def flush(sbase):
            pltpu.async_copy(stage, out_hbm.at[sidx], sem_s).wait()
            for j in range(_STAGE // _L):
                sidx[pl.ds(j * _L, _L)] = dump

        def process_chunk(slab, c_lo, c_w, carry):
            def search_body(j, car):
                qc, sb = car
                w = hits_w[pl.ds(j * _L, _L)]
                b = hits_b[pl.ds(j * _L, _L)]
                wrel = w - c_lo
                lane_ok = j * _L + iota < nh
                m = (wrel >= 0) & (wrel < c_w) & lane_ok

                def append(q):
                    mi = m.astype(jnp.int32)
                    cs = plsc.cumsum(mi)
                    dest = q + cs - mi
                    plsc.store_scatter(qw, [dest], wrel, mask=m)
                    plsc.store_scatter(qb, [dest], b, mask=m)
                    return q + jnp.max(cs)

                qc = lax.cond(jnp.any(m), append, lambda q: q, qc)

                def do_group(car2):
                    qc2, sb2 = car2
                    sb2 = lax.cond(
                        sb2 > _STAGE - _L,
                        lambda s: (flush(s), jnp.int32(0))[1],
                        lambda s: s,
                        sb2,
                    )
                    extract_group(slab, 0, sb2, jnp.full((_L,), True))
                    rem = qw[pl.ds(_L, _L)]
                    qw[pl.ds(0, _L)] = rem
                    remb = qb[pl.ds(_L, _L)]
                    qb[pl.ds(0, _L)] = remb
                    rem2 = qw[pl.ds(2 * _L, _L)]
                    qw[pl.ds(_L, _L)] = rem2
                    remb2 = qb[pl.ds(2 * _L, _L)]
                    qb[pl.ds(_L, _L)] = remb2
                    return (qc2 - _L, sb2 + _L)

                qc, sb = lax.cond(
                    qc >= _L, do_group, lambda car2: car2, (qc, sb)
                )
                return (qc, sb)

            qc, sb = lax.fori_loop(0, nsearch, search_body, carry)

            def drain(car):
                qc2, sb2 = car
                sb2 = lax.cond(
                    sb2 > _STAGE - _L,
                    lambda s: (flush(s), jnp.int32(0))[1],
                    lambda s: s,
                    sb2,
                )
                extract_group(slab, 0, sb2, iota < qc2)
                return (jnp.int32(0), sb2 + qc2)

            qc, sb = lax.cond(qc > 0, drain, lambda car: car, (qc, sb))
            return (qc, sb)

        def fire(c, buf, sem):
            off = pl.multiple_of(lo + c * _CHUNK_W, 128)
            pltpu.async_copy(
                tableT_hbm.at[:, pl.ds(off, _CHUNK_W)],
                buf,
                sem,
            )

        fire(0, slab_a, sem_a)
        fire(1, slab_b, sem_b)

        def chunk_pair(i, carry):
            c = i * 2
            pltpu.make_async_copy(
                tableT_hbm.at[:, pl.ds(0, _CHUNK_W)], slab_a, sem_a
            ).wait()
            carry = process_chunk(slab_a, c * _CHUNK_W, _CHUNK_W, carry)

            @pl.when(i < _NCHUNK // 2 - 1)
            def _():
                fire(c + 2, slab_a, sem_a)

            pltpu.make_async_copy(
                tableT_hbm.at[:, pl.ds(0, _CHUNK_W)], slab_b, sem_b
            ).wait()
            carry = process_chunk(slab_b, (c + 1) * _CHUNK_W, _CHUNK_W, carry)

            @pl.when(i < _NCHUNK // 2 - 1)
            def _():
                fire(c + 3, slab_b, sem_b)

            return carry

        carry = lax.fori_loop(0, _NCHUNK // 2, chunk_pair, (jnp.int32(0), jnp.int32(0)))

        def tail(carry):
            pltpu.sync_copy(tail_hbm, tail_v)
            return process_chunk(tail_v, _COLS * 128, _TAIL_W, carry)

        carry = lax.cond(wid == _NW - 1, tail, lambda car: car, carry)

        _, sb = carry
        lax.cond(sb > 0, lambda s: (flush(s), jnp.int32(0))[1], lambda s: s, sb)

    return fastgather


_KERNEL = _make_kernel()


@jax.jit
def kernel(indices, table):
    tableT = table.T
    out128 = _KERNEL(tableT, indices.astype(jnp.int32), tableT[:, _TAIL_LO:])
    return out128[:BATCH, :EMB_DIM]


# speedup vs baseline: 2.1988x; 1.2706x over previous
import functools

import jax
import jax.numpy as jnp
from jax import lax
from jax.experimental import pallas as pl
from jax.experimental.pallas import tpu as pltpu
from jax.experimental.pallas import tpu_sc as plsc

VOCAB = 1000000
EMB_DIM = 64
BATCH = 16384

_NC = 2
_NW = 32
_L = 16
_BPW = BATCH // _NW
_K = 16
_NT = VOCAB // 8


def _make_kernel():
    mesh = plsc.VectorSubcoreMesh(
        core_axis_name="c", subcore_axis_name="s", num_cores=_NC
    )

    @functools.partial(
        pl.kernel,
        mesh=mesh,
        out_type=jax.ShapeDtypeStruct((EMB_DIM, BATCH), jnp.float32),
        scratch_types=[
            pltpu.VMEM((_BPW + _L,), jnp.int32),
            pltpu.VMEM((EMB_DIM, _BPW), jnp.float32),
        ]
        + [pltpu.VMEM((1, 8, EMB_DIM), jnp.float32) for _ in range(_K)]
        + [pltpu.SemaphoreType.DMA for _ in range(_K)],
        compiler_params=pltpu.CompilerParams(
            use_tc_tiling_on_sc=True, needs_layout_passes=False
        ),
    )
    def tilegather(t3_hbm, idx_hbm, outT_hbm, idx_v, outT_v, *ring_and_sems):
        ring = ring_and_sems[:_K]
        sems = ring_and_sems[_K:]
        wid = lax.axis_index("s") * _NC + lax.axis_index("c")
        base = wid * _BPW
        iota = lax.iota(jnp.int32, _L)

        pltpu.sync_copy(idx_hbm.at[pl.ds(base, _BPW)], idx_v.at[pl.ds(0, _BPW)])

        def fire(w, k):
            pltpu.async_copy(t3_hbm.at[pl.ds(w >> 3, 1)], ring[k], sems[k])

        wv0 = idx_v[pl.ds(0, _L)]
        for k in range(_K):
            fire(wv0[k], k)

        def select(i_s, w, k):
            rv = jnp.full((_L,), 0, jnp.int32) + (w & 7)
            zv = jnp.zeros((_L,), jnp.int32)
            civ = jnp.full((_L,), 0, jnp.int32) + i_s
            for c in range(EMB_DIM // _L):
                ev = c * _L + iota
                vals = plsc.load_gather(ring[k], [zv, rv, ev])
                plsc.store_scatter(outT_v, [ev, civ], vals)

        def block(ib, wv_cur):
            wv_next = idx_v[pl.ds((ib + 1) * _L, _L)]
            for k in range(_K):
                pltpu.make_async_copy(
                    t3_hbm.at[pl.ds(0, 1)], ring[k], sems[k]
                ).wait()
                select(ib * _K + k, wv_cur[k], k)

                @pl.when(ib < _BPW // _K - 1)
                def _():
                    fire(wv_next[k], k)

            return wv_next

        lax.fori_loop(0, _BPW // _K, block, wv0)

        pltpu.sync_copy(outT_v, outT_hbm.at[:, pl.ds(base, _BPW)])

    return tilegather


_KERNEL = _make_kernel()


@jax.jit
def kernel(indices, table):
    t3 = table.reshape(_NT, 8, EMB_DIM)
    outT = _KERNEL(t3, indices.astype(jnp.int32))
    return outT.T
